# baseline (speedup 1.0000x reference)
import jax
import jax.numpy as jnp
from jax.experimental import pallas as pl
from jax.experimental.pallas import tpu as pltpu

_N, _F, _T = 32, 360, 2048
_TT = 2048


_NB = 4


def _ce_body(out_ref, oh_ref, acc_ref):
    x = out_ref[...]
    oh = oh_ref[...]
    l = jnp.log(x + 1e-20)

    s_prod = jnp.sum(oh * l)
    colsum_l = jnp.sum(l, axis=1)
    zero_col = jnp.max(jnp.abs(oh), axis=1) == 0.0
    corr = jnp.sum(jnp.where(zero_col, colsum_l, 0.0))
    acc_ref[0, 0, 0] = s_prod + corr * (1.0 / _F)


def kernel(output, one_hot):
    out = jnp.reshape(output, (_N, _F, _T))
    acc = pl.pallas_call(
        _ce_body,
        grid=(_N // _NB,),
        in_specs=[
            pl.BlockSpec((_NB, _F, _TT), lambda i: (i, 0, 0)),
            pl.BlockSpec((_NB, _F, _TT), lambda i: (i, 0, 0)),
        ],
        out_specs=pl.BlockSpec((1, 1, 1), lambda i: (i, 0, 0),
                               memory_space=pltpu.SMEM),
        out_shape=jax.ShapeDtypeStruct((_N // _NB, 1, 1), jnp.float32),
        compiler_params=pltpu.CompilerParams(
            dimension_semantics=("parallel",),
        ),
    )(out, one_hot)
    return -jnp.sum(acc) / (_T * _N)


# speedup vs baseline: 1.0658x; 1.0658x over previous
import jax
import jax.numpy as jnp
from jax.experimental import pallas as pl
from jax.experimental.pallas import tpu as pltpu

_N, _F, _T = 32, 360, 2048
_TT = 2048


_NB = 2


def _ce_body(out_ref, oh_ref, acc_ref):
    x = out_ref[...]
    oh = oh_ref[...]
    l = jnp.log(x + 1e-20)

    s_prod = jnp.sum(oh * l)
    colsum_l = jnp.sum(l, axis=1)
    zero_col = jnp.max(jnp.abs(oh), axis=1) == 0.0
    corr = jnp.sum(jnp.where(zero_col, colsum_l, 0.0))
    step = s_prod + corr * (1.0 / _F)

    i = pl.program_id(0)

    @pl.when(i == 0)
    def _():
        acc_ref[0, 0] = 0.0

    acc_ref[0, 0] += step

    @pl.when(i == pl.num_programs(0) - 1)
    def _():
        acc_ref[0, 0] = acc_ref[0, 0] * (-1.0 / (_T * _N))


def kernel(output, one_hot):
    out = jnp.reshape(output, (_N, _F, _T))
    acc = pl.pallas_call(
        _ce_body,
        grid=(_N // _NB,),
        in_specs=[
            pl.BlockSpec((_NB, _F, _TT), lambda i: (i, 0, 0)),
            pl.BlockSpec((_NB, _F, _TT), lambda i: (i, 0, 0)),
        ],
        out_specs=pl.BlockSpec((1, 1), lambda i: (0, 0),
                               memory_space=pltpu.SMEM),
        out_shape=jax.ShapeDtypeStruct((1, 1), jnp.float32),
    )(out, one_hot)
    return jnp.reshape(acc, ())


# speedup vs baseline: 1.0876x; 1.0205x over previous
import jax
import jax.numpy as jnp
from jax.experimental import pallas as pl
from jax.experimental.pallas import tpu as pltpu

_N, _F, _T = 32, 360, 2048
_NB = 2


def _ce_body(xa_ref, xb_ref, oha_ref, ohb_ref, acc_ref):
    def contrib(x_ref, oh_ref):
        x = x_ref[...]
        oh = oh_ref[...]
        l = jnp.log(x + 1e-20)
        s_prod = jnp.sum(oh * l)
        colsum_l = jnp.sum(l, axis=1)
        zero_col = jnp.max(jnp.abs(oh), axis=1) == 0.0
        corr = jnp.sum(jnp.where(zero_col, colsum_l, 0.0))
        return s_prod + corr * (1.0 / _F)

    step = contrib(xa_ref, oha_ref) + contrib(xb_ref, ohb_ref)

    i = pl.program_id(0)

    @pl.when(i == 0)
    def _():
        acc_ref[0, 0] = 0.0

    acc_ref[0, 0] += step

    @pl.when(i == pl.num_programs(0) - 1)
    def _():
        acc_ref[0, 0] = acc_ref[0, 0] * (-1.0 / (_T * _N))


def kernel(output, one_hot):
    out = jnp.reshape(output, (_N, _F, _T))
    half_spec_a = pl.BlockSpec((1, _F, _T), lambda i: (2 * i, 0, 0))
    half_spec_b = pl.BlockSpec((1, _F, _T), lambda i: (2 * i + 1, 0, 0))
    acc = pl.pallas_call(
        _ce_body,
        grid=(_N // _NB,),
        in_specs=[half_spec_a, half_spec_b, half_spec_a, half_spec_b],
        out_specs=pl.BlockSpec((1, 1), lambda i: (0, 0),
                               memory_space=pltpu.SMEM),
        out_shape=jax.ShapeDtypeStruct((1, 1), jnp.float32),
    )(out, out, one_hot, one_hot)
    return jnp.reshape(acc, ())
